# initial kernel scaffold (unmeasured)
import jax
import jax.numpy as jnp
from jax import lax
from jax.experimental import pallas as pl
from jax.experimental.pallas import tpu as pltpu

N_DEV = 4
M, K, N = 4096, 4096, 8192
CH = M // N_DEV
T = 256

_GELU_C = 0.7978845608028654


def _gelu(y):
    return 0.5 * y * (1.0 + jnp.tanh(_GELU_C * (y + 0.044715 * y * y * y)))


def kernel(x, w_mat):
    partial = jnp.dot(x, w_mat, preferred_element_type=jnp.float32)

    def body(p_ref, out_ref, recv_ref, acc_ref, va, vb, copy_sem,
             rs_send_sems, rs_recv_sems, ag_send_sems, ag_recv_sems):
        d = lax.axis_index("i")
        left = (d - 1) % N_DEV
        right = (d + 1) % N_DEV

        barrier = pltpu.get_barrier_semaphore()
        for nbr in (left, right):
            pl.semaphore_signal(barrier, inc=1, device_id=(nbr,),
                                device_id_type=pl.DeviceIdType.MESH)
        pl.semaphore_wait(barrier, 2)

        def add_tiles(src_a_hbm, src_b_hbm, dst_hbm, fuse_gelu):
            for t in range(CH // T):
                rows = pl.ds(t * T, T)
                cp = pltpu.make_async_copy(src_a_hbm.at[rows, :], va, copy_sem)
                cp.start()
                cp.wait()
                cp = pltpu.make_async_copy(src_b_hbm.at[rows, :], vb, copy_sem)
                cp.start()
                cp.wait()
                s = va[...] + vb[...]
                va[...] = _gelu(s) if fuse_gelu else s
                cp = pltpu.make_async_copy(va, dst_hbm.at[rows, :], copy_sem)
                cp.start()
                cp.wait()

        for s in range(N_DEV - 1):
            c_send = (d - s) % N_DEV
            if s == 0:
                src = p_ref.at[pl.ds(c_send * CH, CH), :]
            else:
                add_tiles(recv_ref.at[s - 1],
                          p_ref.at[pl.ds(c_send * CH, CH), :],
                          acc_ref, fuse_gelu=False)
                src = acc_ref
            rdma = pltpu.make_async_remote_copy(
                src_ref=src,
                dst_ref=recv_ref.at[s],
                send_sem=rs_send_sems.at[s],
                recv_sem=rs_recv_sems.at[s],
                device_id=(right,),
                device_id_type=pl.DeviceIdType.MESH,
            )
            rdma.start()
            rdma.wait()

        o = (d + 1) % N_DEV
        add_tiles(recv_ref.at[N_DEV - 2],
                  p_ref.at[pl.ds(o * CH, CH), :],
                  out_ref.at[pl.ds(o * CH, CH), :], fuse_gelu=True)

        for s in range(N_DEV - 1):
            c_send = (o - s) % N_DEV
            c_recv = (d - s) % N_DEV
            send = pltpu.make_async_remote_copy(
                src_ref=out_ref.at[pl.ds(c_send * CH, CH), :],
                dst_ref=out_ref.at[pl.ds(c_send * CH, CH), :],
                send_sem=ag_send_sems.at[s],
                recv_sem=ag_recv_sems.at[s],
                device_id=(right,),
                device_id_type=pl.DeviceIdType.MESH,
            )
            send.start()
            send.wait_send()
            recv = pltpu.make_async_remote_copy(
                src_ref=out_ref.at[pl.ds(c_recv * CH, CH), :],
                dst_ref=out_ref.at[pl.ds(c_recv * CH, CH), :],
                send_sem=ag_send_sems.at[s],
                recv_sem=ag_recv_sems.at[s],
                device_id=(left,),
                device_id_type=pl.DeviceIdType.MESH,
            )
            recv.wait_recv()

    return pl.pallas_call(
        body,
        out_shape=jax.ShapeDtypeStruct((M, N), jnp.float32),
        in_specs=[pl.BlockSpec(memory_space=pltpu.ANY)],
        out_specs=pl.BlockSpec(memory_space=pltpu.ANY),
        scratch_shapes=[
            pltpu.ANY((N_DEV - 1, CH, N), jnp.float32),
            pltpu.ANY((CH, N), jnp.float32),
            pltpu.VMEM((T, N), jnp.float32),
            pltpu.VMEM((T, N), jnp.float32),
            pltpu.SemaphoreType.DMA,
            pltpu.SemaphoreType.DMA((N_DEV - 1,)),
            pltpu.SemaphoreType.DMA((N_DEV - 1,)),
            pltpu.SemaphoreType.DMA((N_DEV - 1,)),
            pltpu.SemaphoreType.DMA((N_DEV - 1,)),
        ],
        compiler_params=pltpu.CompilerParams(collective_id=0),
    )(partial)


# baseline (device time: 2482970 ns/iter reference)
import jax
import jax.numpy as jnp
from jax import lax
from jax.experimental import pallas as pl
from jax.experimental.pallas import tpu as pltpu

N_DEV = 4
M, K, N = 4096, 4096, 8192
CH = M // N_DEV
T = 256

_GELU_C = 0.7978845608028654


def _gelu(y):
    return 0.5 * y * (1.0 + jnp.tanh(_GELU_C * (y + 0.044715 * y * y * y)))


def kernel(x, w_mat):
    partial = jnp.dot(x, w_mat, preferred_element_type=jnp.float32)

    def body(p_ref, out_ref, recv_ref, acc_ref, va, vb, copy_sem,
             rs_send_sems, rs_recv_sems, ag_send_sems, ag_recv_sems):
        d = lax.axis_index("i")
        left = (d - 1) % N_DEV
        right = (d + 1) % N_DEV

        barrier = pltpu.get_barrier_semaphore()
        for nbr in (left, right):
            pl.semaphore_signal(barrier, inc=1, device_id=(nbr,),
                                device_id_type=pl.DeviceIdType.MESH)
        pl.semaphore_wait(barrier, 2)

        def add_tiles(src_a_hbm, src_b_hbm, dst_hbm, fuse_gelu):
            for t in range(CH // T):
                rows = pl.ds(t * T, T)
                cp = pltpu.make_async_copy(src_a_hbm.at[rows, :], va, copy_sem)
                cp.start()
                cp.wait()
                cp = pltpu.make_async_copy(src_b_hbm.at[rows, :], vb, copy_sem)
                cp.start()
                cp.wait()
                s = va[...] + vb[...]
                va[...] = _gelu(s) if fuse_gelu else s
                cp = pltpu.make_async_copy(va, dst_hbm.at[rows, :], copy_sem)
                cp.start()
                cp.wait()

        for s in range(N_DEV - 1):
            c_send = (d - s) % N_DEV
            if s == 0:
                src = p_ref.at[pl.ds(c_send * CH, CH), :]
            else:
                add_tiles(recv_ref.at[s - 1],
                          p_ref.at[pl.ds(c_send * CH, CH), :],
                          acc_ref, fuse_gelu=False)
                src = acc_ref
            rdma = pltpu.make_async_remote_copy(
                src_ref=src,
                dst_ref=recv_ref.at[s],
                send_sem=rs_send_sems.at[s],
                recv_sem=rs_recv_sems.at[s],
                device_id=(right,),
                device_id_type=pl.DeviceIdType.MESH,
            )
            rdma.start()
            rdma.wait()

        o = (d + 1) % N_DEV
        add_tiles(recv_ref.at[N_DEV - 2],
                  p_ref.at[pl.ds(o * CH, CH), :],
                  out_ref.at[pl.ds(o * CH, CH), :], fuse_gelu=True)

        for s in range(N_DEV - 1):
            c_send = (o - s) % N_DEV
            c_recv = (d - s) % N_DEV
            send = pltpu.make_async_remote_copy(
                src_ref=out_ref.at[pl.ds(c_send * CH, CH), :],
                dst_ref=out_ref.at[pl.ds(c_send * CH, CH), :],
                send_sem=ag_send_sems.at[s],
                recv_sem=ag_recv_sems.at[s],
                device_id=(right,),
                device_id_type=pl.DeviceIdType.MESH,
            )
            send.start()
            send.wait_send()
            recv = pltpu.make_async_remote_copy(
                src_ref=out_ref.at[pl.ds(c_recv * CH, CH), :],
                dst_ref=out_ref.at[pl.ds(c_recv * CH, CH), :],
                send_sem=ag_send_sems.at[s],
                recv_sem=ag_recv_sems.at[s],
                device_id=(left,),
                device_id_type=pl.DeviceIdType.MESH,
            )
            recv.wait_recv()

    out, _, _ = pl.pallas_call(
        body,
        out_shape=[
            jax.ShapeDtypeStruct((M, N), jnp.float32),
            jax.ShapeDtypeStruct((N_DEV - 1, CH, N), jnp.float32),
            jax.ShapeDtypeStruct((CH, N), jnp.float32),
        ],
        in_specs=[pl.BlockSpec(memory_space=pl.ANY)],
        out_specs=[
            pl.BlockSpec(memory_space=pl.ANY),
            pl.BlockSpec(memory_space=pl.ANY),
            pl.BlockSpec(memory_space=pl.ANY),
        ],
        scratch_shapes=[
            pltpu.VMEM((T, N), jnp.float32),
            pltpu.VMEM((T, N), jnp.float32),
            pltpu.SemaphoreType.DMA,
            pltpu.SemaphoreType.DMA((N_DEV - 1,)),
            pltpu.SemaphoreType.DMA((N_DEV - 1,)),
            pltpu.SemaphoreType.DMA((N_DEV - 1,)),
            pltpu.SemaphoreType.DMA((N_DEV - 1,)),
        ],
        compiler_params=pltpu.CompilerParams(collective_id=0),
    )(partial)
    return out


# device time: 1405662 ns/iter; 1.7664x vs baseline; 1.7664x over previous
import jax
import jax.numpy as jnp
from jax import lax
from jax.experimental import pallas as pl
from jax.experimental.pallas import tpu as pltpu

N_DEV = 4
M, K, N = 4096, 4096, 8192
CH = M // N_DEV
NH = N // 2
T = 512

_GELU_C = 0.7978845608028654


def _gelu(y):
    return 0.5 * y * (1.0 + jnp.tanh(_GELU_C * (y + 0.044715 * y * y * y)))


def kernel(x, w_mat):
    partial = jnp.dot(x, w_mat, preferred_element_type=jnp.float32)

    def body(p_ref, out_ref, recv_ref, acc_ref, va, vb, copy_sem,
             rs_send_sems, rs_recv_sems, ag_send_sems, ag_recv_sems):
        d = lax.axis_index("i")
        left = (d - 1) % N_DEV
        right = (d + 1) % N_DEV

        barrier = pltpu.get_barrier_semaphore()
        for nbr in (left, right):
            pl.semaphore_signal(barrier, inc=1, device_id=(nbr,),
                                device_id_type=pl.DeviceIdType.MESH)
        pl.semaphore_wait(barrier, 2)

        def cols(r):
            return pl.ds(r * NH, NH)

        def chunk_send(r, s):
            return ((d - s) if r == 0 else (d + s)) % N_DEV

        def owner(r):
            return ((d + 1) if r == 0 else (d - 1)) % N_DEV

        def peer(r):
            return right if r == 0 else left

        def add_tiles(src_a_hbm, src_b_hbm, dst_hbm, fuse_gelu):
            for t in range(CH // T):
                rows = pl.ds(t * T, T)
                cp = pltpu.make_async_copy(src_a_hbm.at[rows, :], va, copy_sem)
                cp.start()
                cp.wait()
                cp = pltpu.make_async_copy(src_b_hbm.at[rows, :], vb, copy_sem)
                cp.start()
                cp.wait()
                s = va[...] + vb[...]
                va[...] = _gelu(s) if fuse_gelu else s
                cp = pltpu.make_async_copy(va, dst_hbm.at[rows, :], copy_sem)
                cp.start()
                cp.wait()

        for s in range(N_DEV - 1):
            rdmas = []
            for r in (0, 1):
                c = chunk_send(r, s)
                if s == 0:
                    src = p_ref.at[pl.ds(c * CH, CH), cols(r)]
                else:
                    add_tiles(recv_ref.at[r, s - 1],
                              p_ref.at[pl.ds(c * CH, CH), cols(r)],
                              acc_ref.at[r], fuse_gelu=False)
                    src = acc_ref.at[r]
                rdma = pltpu.make_async_remote_copy(
                    src_ref=src,
                    dst_ref=recv_ref.at[r, s],
                    send_sem=rs_send_sems.at[r, s],
                    recv_sem=rs_recv_sems.at[r, s],
                    device_id=(peer(r),),
                    device_id_type=pl.DeviceIdType.MESH,
                )
                rdma.start()
                rdmas.append(rdma)
            for rdma in rdmas:
                rdma.wait()

        for r in (0, 1):
            o = owner(r)
            add_tiles(recv_ref.at[r, N_DEV - 2],
                      p_ref.at[pl.ds(o * CH, CH), cols(r)],
                      out_ref.at[pl.ds(o * CH, CH), cols(r)], fuse_gelu=True)

        for s in range(N_DEV - 1):
            sends = []
            recvs = []
            for r in (0, 1):
                c_send = ((owner(r) - s) if r == 0 else (owner(r) + s)) % N_DEV
                c_recv = ((d - s) if r == 0 else (d + s)) % N_DEV
                send = pltpu.make_async_remote_copy(
                    src_ref=out_ref.at[pl.ds(c_send * CH, CH), cols(r)],
                    dst_ref=out_ref.at[pl.ds(c_send * CH, CH), cols(r)],
                    send_sem=ag_send_sems.at[r, s],
                    recv_sem=ag_recv_sems.at[r, s],
                    device_id=(peer(r),),
                    device_id_type=pl.DeviceIdType.MESH,
                )
                send.start()
                sends.append(send)
                recv = pltpu.make_async_remote_copy(
                    src_ref=out_ref.at[pl.ds(c_recv * CH, CH), cols(r)],
                    dst_ref=out_ref.at[pl.ds(c_recv * CH, CH), cols(r)],
                    send_sem=ag_send_sems.at[r, s],
                    recv_sem=ag_recv_sems.at[r, s],
                    device_id=(peer(r),),
                    device_id_type=pl.DeviceIdType.MESH,
                )
                recvs.append(recv)
            for send in sends:
                send.wait_send()
            for recv in recvs:
                recv.wait_recv()

    out, _, _ = pl.pallas_call(
        body,
        out_shape=[
            jax.ShapeDtypeStruct((M, N), jnp.float32),
            jax.ShapeDtypeStruct((2, N_DEV - 1, CH, NH), jnp.float32),
            jax.ShapeDtypeStruct((2, CH, NH), jnp.float32),
        ],
        in_specs=[pl.BlockSpec(memory_space=pl.ANY)],
        out_specs=[
            pl.BlockSpec(memory_space=pl.ANY),
            pl.BlockSpec(memory_space=pl.ANY),
            pl.BlockSpec(memory_space=pl.ANY),
        ],
        scratch_shapes=[
            pltpu.VMEM((T, NH), jnp.float32),
            pltpu.VMEM((T, NH), jnp.float32),
            pltpu.SemaphoreType.DMA,
            pltpu.SemaphoreType.DMA((2, N_DEV - 1)),
            pltpu.SemaphoreType.DMA((2, N_DEV - 1)),
            pltpu.SemaphoreType.DMA((2, N_DEV - 1)),
            pltpu.SemaphoreType.DMA((2, N_DEV - 1)),
        ],
        compiler_params=pltpu.CompilerParams(collective_id=0),
    )(partial)
    return out


# device time: 1252572 ns/iter; 1.9823x vs baseline; 1.1222x over previous
import jax
import jax.numpy as jnp
from jax import lax
from jax.experimental import pallas as pl
from jax.experimental.pallas import tpu as pltpu

N_DEV = 4
M, K, N = 4096, 4096, 8192
CH = M // N_DEV
NH = N // 2
B = 4
SUB = CH // B

_GELU_C = 0.7978845608028654


def _gelu(y):
    return 0.5 * y * (1.0 + jnp.tanh(_GELU_C * (y + 0.044715 * y * y * y)))


def kernel(x, w_mat):
    partial = jnp.dot(x, w_mat, preferred_element_type=jnp.float32)

    def body(p_ref, out_ref, recv_ref, va, vb, copy_sems,
             rs_send_sems, rs_recv_sems, ag_send_sems, ag_recv_sems):
        d = lax.axis_index("i")
        left = (d - 1) % N_DEV
        right = (d + 1) % N_DEV

        barrier = pltpu.get_barrier_semaphore()
        for nbr in (left, right):
            pl.semaphore_signal(barrier, inc=1, device_id=(nbr,),
                                device_id_type=pl.DeviceIdType.MESH)
        pl.semaphore_wait(barrier, 2)

        def cols(r):
            return pl.ds(r * NH, NH)

        def sub_rows(c, k):
            return pl.ds(c * CH + k * SUB, SUB)

        def chunk_send(r, s):
            return ((d - s) if r == 0 else (d + s)) % N_DEV

        def owner(r):
            return ((d + 1) if r == 0 else (d - 1)) % N_DEV

        def peer(r):
            return right if r == 0 else left

        pending_send = {}
        pending_store = {}

        def wait_buf(r, q):
            rd = pending_send.pop((r, q), None)
            if rd is not None:
                rd.wait_send()
            cp = pending_store.pop((r, q), None)
            if cp is not None:
                cp.wait()

        end_waits = []

        def rs_wait_recv(r, s, k):
            rd = pltpu.make_async_remote_copy(
                src_ref=recv_ref.at[r, s, pl.ds(k * SUB, SUB), :],
                dst_ref=recv_ref.at[r, s, pl.ds(k * SUB, SUB), :],
                send_sem=rs_send_sems.at[r, s, k],
                recv_sem=rs_recv_sems.at[r, s, k],
                device_id=(peer(r),),
                device_id_type=pl.DeviceIdType.MESH,
            )
            rd.wait_recv()

        for k in range(B):
            for r in (0, 1):
                c = chunk_send(r, 0)
                rd = pltpu.make_async_remote_copy(
                    src_ref=p_ref.at[sub_rows(c, k), cols(r)],
                    dst_ref=recv_ref.at[r, 0, pl.ds(k * SUB, SUB), :],
                    send_sem=rs_send_sems.at[r, 0, k],
                    recv_sem=rs_recv_sems.at[r, 0, k],
                    device_id=(peer(r),),
                    device_id_type=pl.DeviceIdType.MESH,
                )
                rd.start()
                end_waits.append(rd)

        for s in range(1, N_DEV - 1):
            for k in range(B):
                q = k % 2
                for r in (0, 1):
                    c = chunk_send(r, s)
                    rs_wait_recv(r, s - 1, k)
                    wait_buf(r, q)
                    cp_a = pltpu.make_async_copy(
                        recv_ref.at[r, s - 1, pl.ds(k * SUB, SUB), :],
                        va.at[r, q], copy_sems.at[r, q, 0])
                    cp_b = pltpu.make_async_copy(
                        p_ref.at[sub_rows(c, k), cols(r)],
                        vb.at[r, q], copy_sems.at[r, q, 1])
                    cp_a.start()
                    cp_b.start()
                    cp_a.wait()
                    cp_b.wait()
                    va[r, q] = va[r, q] + vb[r, q]
                    rd = pltpu.make_async_remote_copy(
                        src_ref=va.at[r, q],
                        dst_ref=recv_ref.at[r, s, pl.ds(k * SUB, SUB), :],
                        send_sem=rs_send_sems.at[r, s, k],
                        recv_sem=rs_recv_sems.at[r, s, k],
                        device_id=(peer(r),),
                        device_id_type=pl.DeviceIdType.MESH,
                    )
                    rd.start()
                    pending_send[(r, q)] = rd

        for k in range(B):
            q = k % 2
            for r in (0, 1):
                o = owner(r)
                rs_wait_recv(r, N_DEV - 2, k)
                wait_buf(r, q)
                cp_a = pltpu.make_async_copy(
                    recv_ref.at[r, N_DEV - 2, pl.ds(k * SUB, SUB), :],
                    va.at[r, q], copy_sems.at[r, q, 0])
                cp_b = pltpu.make_async_copy(
                    p_ref.at[sub_rows(o, k), cols(r)],
                    vb.at[r, q], copy_sems.at[r, q, 1])
                cp_a.start()
                cp_b.start()
                cp_a.wait()
                cp_b.wait()
                va[r, q] = _gelu(va[r, q] + vb[r, q])
                rd = pltpu.make_async_remote_copy(
                    src_ref=va.at[r, q],
                    dst_ref=out_ref.at[sub_rows(o, k), cols(r)],
                    send_sem=ag_send_sems.at[r, 0, k],
                    recv_sem=ag_recv_sems.at[r, 0, k],
                    device_id=(peer(r),),
                    device_id_type=pl.DeviceIdType.MESH,
                )
                rd.start()
                pending_send[(r, q)] = rd
                cp = pltpu.make_async_copy(
                    va.at[r, q], out_ref.at[sub_rows(o, k), cols(r)],
                    copy_sems.at[r, q, 2])
                cp.start()
                pending_store[(r, q)] = cp

        for s in range(1, N_DEV - 1):
            for k in range(B):
                for r in (0, 1):
                    c_prev = ((d - s + 1) if r == 0 else (d + s - 1)) % N_DEV
                    rd = pltpu.make_async_remote_copy(
                        src_ref=out_ref.at[sub_rows(c_prev, k), cols(r)],
                        dst_ref=out_ref.at[sub_rows(c_prev, k), cols(r)],
                        send_sem=ag_send_sems.at[r, s - 1, k],
                        recv_sem=ag_recv_sems.at[r, s - 1, k],
                        device_id=(peer(r),),
                        device_id_type=pl.DeviceIdType.MESH,
                    )
                    rd.wait_recv()
                    fw = pltpu.make_async_remote_copy(
                        src_ref=out_ref.at[sub_rows(c_prev, k), cols(r)],
                        dst_ref=out_ref.at[sub_rows(c_prev, k), cols(r)],
                        send_sem=ag_send_sems.at[r, s, k],
                        recv_sem=ag_recv_sems.at[r, s, k],
                        device_id=(peer(r),),
                        device_id_type=pl.DeviceIdType.MESH,
                    )
                    fw.start()
                    end_waits.append(fw)

        for k in range(B):
            for r in (0, 1):
                c_last = ((d - N_DEV + 2) if r == 0 else (d + N_DEV - 2)) % N_DEV
                rd = pltpu.make_async_remote_copy(
                    src_ref=out_ref.at[sub_rows(c_last, k), cols(r)],
                    dst_ref=out_ref.at[sub_rows(c_last, k), cols(r)],
                    send_sem=ag_send_sems.at[r, N_DEV - 2, k],
                    recv_sem=ag_recv_sems.at[r, N_DEV - 2, k],
                    device_id=(peer(r),),
                    device_id_type=pl.DeviceIdType.MESH,
                )
                rd.wait_recv()
        for rd in end_waits:
            rd.wait_send()
        for rd in pending_send.values():
            rd.wait_send()
        for cp in pending_store.values():
            cp.wait()

    out, _ = pl.pallas_call(
        body,
        out_shape=[
            jax.ShapeDtypeStruct((M, N), jnp.float32),
            jax.ShapeDtypeStruct((2, N_DEV - 1, CH, NH), jnp.float32),
        ],
        in_specs=[pl.BlockSpec(memory_space=pl.ANY)],
        out_specs=[
            pl.BlockSpec(memory_space=pl.ANY),
            pl.BlockSpec(memory_space=pl.ANY),
        ],
        scratch_shapes=[
            pltpu.VMEM((2, 2, SUB, NH), jnp.float32),
            pltpu.VMEM((2, 2, SUB, NH), jnp.float32),
            pltpu.SemaphoreType.DMA((2, 2, 3)),
            pltpu.SemaphoreType.DMA((2, N_DEV - 1, B)),
            pltpu.SemaphoreType.DMA((2, N_DEV - 1, B)),
            pltpu.SemaphoreType.DMA((2, N_DEV - 1, B)),
            pltpu.SemaphoreType.DMA((2, N_DEV - 1, B)),
        ],
        compiler_params=pltpu.CompilerParams(collective_id=0),
    )(partial)
    return out
